# baseline (speedup 1.0000x reference)
import functools

import jax
import jax.numpy as jnp
from jax import lax
from jax.experimental import pallas as pl
from jax.experimental.pallas import tpu as pltpu
from jax.experimental.pallas import tpu_sc as plsc

_NBUF = 4


def _make_copy_kernel(V, D):
    info = plsc.get_sparse_core_info()
    num_workers = info.num_cores * info.num_subcores
    rows_per_w = V // num_workers
    sub = rows_per_w // _NBUF
    mesh = plsc.VectorSubcoreMesh(core_axis_name="c", subcore_axis_name="s")

    @functools.partial(
        pl.kernel,
        mesh=mesh,
        out_type=jax.ShapeDtypeStruct((V, D), jnp.float32),
        scratch_types=[pltpu.VMEM((_NBUF, sub, D), jnp.float32)]
        + [pltpu.SemaphoreType.DMA] * (2 * _NBUF),
    )
    def copy_k(table_hbm, out_hbm, buf, *sems):
        wid = lax.axis_index("s") * info.num_cores + lax.axis_index("c")
        base = wid * rows_per_w
        reads = [
            pltpu.async_copy(
                table_hbm.at[pl.ds(base + j * sub, sub)], buf.at[j], sems[j]
            )
            for j in range(_NBUF)
        ]
        writes = []
        for j in range(_NBUF):
            reads[j].wait()
            writes.append(
                pltpu.async_copy(
                    buf.at[j],
                    out_hbm.at[pl.ds(base + j * sub, sub)],
                    sems[_NBUF + j],
                )
            )
        for w in writes:
            w.wait()

    return copy_k


def kernel(seq_length, table):
    V, D = table.shape
    out = _make_copy_kernel(V, D)(table)
    return out[None, :, :]
